# baseline (speedup 1.0000x reference)
import functools

import jax
import jax.numpy as jnp
from jax import lax
from jax.experimental import pallas as pl
from jax.experimental.pallas import tpu as pltpu
from jax.experimental.pallas import tpu_sc as plsc

N = 50000
H = 32
D_IN = 128
G = 64
D_OUT = 10

BLK = 1024
NBLK = 49
NP = BLK * NBLK
ROWS_PER_TILE_NODES = NP // 16

NC = 2
NS = 16
SUB = 128
KSUB = 8
CHUNK = SUB * KSUB
NCHUNK = 50
EPAD = NC * NS * NCHUNK * CHUNK
EROWS = EPAD // SUB


def _mesh():
    return plsc.VectorSubcoreMesh(core_axis_name="c", subcore_axis_name="s")


_SC_PARAMS = pltpu.CompilerParams(use_tc_tiling_on_sc=False)


DEGW = 16
DEG_RPI = 8
DEG_ITERS = EROWS // (NC * NS) // DEG_RPI


def _deg_body(col2d, z16, ones_h, out, colv, ones_v, sem, acc):
    cid = lax.axis_index("c")
    sid = lax.axis_index("s")
    r0 = sid * ROWS_PER_TILE_NODES
    pltpu.sync_copy(z16.at[pl.ds(r0, ROWS_PER_TILE_NODES)],
                    acc.at[pl.ds(r0, ROWS_PER_TILE_NODES)])
    pltpu.sync_copy(ones_h, ones_v)
    plsc.subcore_barrier()

    base = (cid * NS + sid) * DEG_ITERS * DEG_RPI

    def body(i, carry):
        off = base + i * DEG_RPI
        pltpu.sync_copy(col2d.at[pl.ds(off, DEG_RPI)], colv)
        for j in range(DEG_RPI):
            pltpu.async_copy(ones_v, acc.at[colv.at[j]], sem, add=True)
        for j in range(DEG_RPI):
            pltpu.make_async_copy(ones_v, acc.at[colv.at[j]], sem).wait()
        return carry

    lax.fori_loop(0, DEG_ITERS, body, 0)
    plsc.subcore_barrier()
    pltpu.sync_copy(acc.at[pl.ds(r0, ROWS_PER_TILE_NODES)],
                    out.at[cid, pl.ds(r0, ROWS_PER_TILE_NODES)])


def _deg_call(col2d, z16, ones_h):
    return pl.kernel(
        _deg_body,
        out_type=jax.ShapeDtypeStruct((NC, NP, DEGW), jnp.float32),
        mesh=_mesh(),
        scratch_types=[
            pltpu.VMEM((DEG_RPI, SUB), jnp.int32),
            pltpu.VMEM((SUB, DEGW), jnp.float32),
            pltpu.SemaphoreType.DMA,
            pltpu.VMEM_SHARED((NP, DEGW), jnp.float32),
        ],
        compiler_params=_SC_PARAMS,
    )(col2d, z16, ones_h)


NPAIR = NCHUNK // 2


def _edge_body(tp, rc3, z2, out,
               idxA, idxB, msgA, msgB, semA, semB, semSA, semSB, acc):
    cid = lax.axis_index("c")
    sid = lax.axis_index("s")
    r0 = sid * ROWS_PER_TILE_NODES
    pltpu.sync_copy(z2.at[pl.ds(r0, ROWS_PER_TILE_NODES)],
                    acc.at[pl.ds(r0, ROWS_PER_TILE_NODES)])
    plsc.subcore_barrier()

    base = (cid * NS + sid) * NCHUNK

    def fire_g(idx, msg, sem):
        for j in range(KSUB):
            pltpu.async_copy(tp.at[idx.at[j]], msg.at[j], sem)

    def drain_g(idx, msg, sem):
        for j in range(KSUB):
            pltpu.make_async_copy(tp.at[idx.at[j]], msg.at[j], sem).wait()

    def fire_s(idx, msg, sem):
        for j in range(KSUB):
            pltpu.async_copy(msg.at[j], acc.at[idx.at[KSUB + j]], sem,
                             add=True)

    def drain_s(idx, msg, sem):
        for j in range(KSUB):
            pltpu.make_async_copy(msg.at[j], acc.at[idx.at[KSUB + j]],
                                  sem).wait()

    pltpu.sync_copy(rc3.at[base], idxA)
    fire_g(idxA, msgA, semA)

    def pair(u, carry):
        c1 = base + 2 * u + 1
        pltpu.sync_copy(rc3.at[c1], idxB)
        fire_g(idxB, msgB, semB)
        drain_g(idxA, msgA, semA)
        fire_s(idxA, msgA, semSA)
        drain_s(idxA, msgA, semSA)
        pltpu.sync_copy(rc3.at[c1 + 1], idxA)
        fire_g(idxA, msgA, semA)
        drain_g(idxB, msgB, semB)
        fire_s(idxB, msgB, semSB)
        drain_s(idxB, msgB, semSB)
        return carry

    lax.fori_loop(0, NPAIR - 1, pair, 0)

    c1 = base + NCHUNK - 1
    pltpu.sync_copy(rc3.at[c1], idxB)
    fire_g(idxB, msgB, semB)
    drain_g(idxA, msgA, semA)
    fire_s(idxA, msgA, semSA)
    drain_s(idxA, msgA, semSA)
    drain_g(idxB, msgB, semB)
    fire_s(idxB, msgB, semSB)
    drain_s(idxB, msgB, semSB)

    plsc.subcore_barrier()
    pltpu.sync_copy(acc.at[pl.ds(r0, ROWS_PER_TILE_NODES)],
                    out.at[cid, pl.ds(r0, ROWS_PER_TILE_NODES)])


def _edge_call(tp, rc3, z2):
    return pl.kernel(
        _edge_body,
        out_type=jax.ShapeDtypeStruct((NC, NP, H), jnp.bfloat16),
        mesh=_mesh(),
        scratch_types=[
            pltpu.VMEM((2 * KSUB, SUB), jnp.int32),
            pltpu.VMEM((2 * KSUB, SUB), jnp.int32),
            pltpu.VMEM((KSUB, SUB, H), jnp.bfloat16),
            pltpu.VMEM((KSUB, SUB, H), jnp.bfloat16),
            pltpu.SemaphoreType.DMA,
            pltpu.SemaphoreType.DMA,
            pltpu.SemaphoreType.DMA,
            pltpu.SemaphoreType.DMA,
            pltpu.VMEM_SHARED((NP, H), jnp.bfloat16),
        ],
        compiler_params=_SC_PARAMS,
    )(tp, rc3, z2)


def _dinv(deg_ref):
    deg = deg_ref[0, :] + deg_ref[1, :]
    return jnp.where(deg > 0, lax.rsqrt(deg), 0.0)


def _first_body(x_ref, w_ref, v_ref, b_ref, t_ref, r_ref):
    x = x_ref[...]
    t_ref[...] = jnp.dot(x, w_ref[...], preferred_element_type=jnp.float32)
    r_ref[...] = jnp.dot(x, v_ref[...],
                         preferred_element_type=jnp.float32) + b_ref[...]


def _first_call(x, w, v, b):
    return pl.pallas_call(
        _first_body,
        grid=(NBLK,),
        in_specs=[
            pl.BlockSpec((BLK, D_IN), lambda i: (i, 0)),
            pl.BlockSpec((D_IN, H), lambda i: (0, 0)),
            pl.BlockSpec((D_IN, H), lambda i: (0, 0)),
            pl.BlockSpec((1, H), lambda i: (0, 0)),
        ],
        out_specs=[
            pl.BlockSpec((BLK, H), lambda i: (i, 0)),
            pl.BlockSpec((BLK, H), lambda i: (i, 0)),
        ],
        out_shape=[
            jax.ShapeDtypeStruct((NP, H), jnp.float32),
            jax.ShapeDtypeStruct((NP, H), jnp.float32),
        ],
    )(x, w, v, b)


def _scale_body(t_ref, deg_ref, tp_ref):
    dinv = _dinv(deg_ref)
    tp_ref[...] = (t_ref[...] * dinv[:, None]).astype(jnp.bfloat16)


def _scale_call(t, deg2):
    return pl.pallas_call(
        _scale_body,
        grid=(NBLK,),
        in_specs=[
            pl.BlockSpec((BLK, H), lambda i: (i, 0)),
            pl.BlockSpec((NC, BLK), lambda i: (0, i)),
        ],
        out_specs=pl.BlockSpec((BLK, H), lambda i: (i, 0)),
        out_shape=jax.ShapeDtypeStruct((NP, H), jnp.bfloat16),
    )(t, deg2)


def _mid_body(s_ref, deg_ref, rp_ref, w_ref, v_ref, b_ref, tp_ref, r_ref):
    dinv = _dinv(deg_ref)
    s = s_ref[0].astype(jnp.float32) + s_ref[1].astype(jnp.float32)
    h = jnp.maximum(dinv[:, None] * s + rp_ref[...], 0.0)
    t = jnp.dot(h, w_ref[...], preferred_element_type=jnp.float32)
    tp_ref[...] = (t * dinv[:, None]).astype(jnp.bfloat16)
    r_ref[...] = jnp.dot(h, v_ref[...],
                         preferred_element_type=jnp.float32) + b_ref[...]


def _mid_call(s, deg2, rp, w, v, b):
    return pl.pallas_call(
        _mid_body,
        grid=(NBLK,),
        in_specs=[
            pl.BlockSpec((NC, BLK, H), lambda i: (0, i, 0)),
            pl.BlockSpec((NC, BLK), lambda i: (0, i)),
            pl.BlockSpec((BLK, H), lambda i: (i, 0)),
            pl.BlockSpec((H, H), lambda i: (0, 0)),
            pl.BlockSpec((H, H), lambda i: (0, 0)),
            pl.BlockSpec((1, H), lambda i: (0, 0)),
        ],
        out_specs=[
            pl.BlockSpec((BLK, H), lambda i: (i, 0)),
            pl.BlockSpec((BLK, H), lambda i: (i, 0)),
        ],
        out_shape=[
            jax.ShapeDtypeStruct((NP, H), jnp.bfloat16),
            jax.ShapeDtypeStruct((NP, H), jnp.float32),
        ],
    )(s, deg2, rp, w, v, b)


def _final_body(s_ref, deg_ref, rp_ref, batch_ref, wd_ref, bd_ref, out_ref,
                sums, cnt):
    i = pl.program_id(0)

    @pl.when(i == 0)
    def _():
        sums[...] = jnp.zeros_like(sums)
        cnt[...] = jnp.zeros_like(cnt)

    dinv = _dinv(deg_ref)
    s = s_ref[0].astype(jnp.float32) + s_ref[1].astype(jnp.float32)
    h = jnp.maximum(dinv[:, None] * s + rp_ref[...], 0.0)
    b = batch_ref[0, 0, :]
    onehot = (b[:, None] == lax.broadcasted_iota(jnp.int32, (BLK, G), 1)
              ).astype(jnp.float32)
    sums[...] += lax.dot_general(onehot, h, (((0,), (0,)), ((), ())),
                                 preferred_element_type=jnp.float32)
    csum = jnp.sum(onehot, axis=0)
    cnt[...] += jnp.broadcast_to(csum[:, None], cnt.shape)

    @pl.when(i == NBLK - 1)
    def _():
        pooled = sums[...] / jnp.maximum(cnt[...], 1.0)
        logits = jnp.dot(pooled, wd_ref[...],
                         preferred_element_type=jnp.float32) + bd_ref[...]
        out_ref[...] = jax.nn.sigmoid(logits)


def _final_call(s, deg2, rp, batch3, wd, bd):
    return pl.pallas_call(
        _final_body,
        grid=(NBLK,),
        in_specs=[
            pl.BlockSpec((NC, BLK, H), lambda i: (0, i, 0)),
            pl.BlockSpec((NC, BLK), lambda i: (0, i)),
            pl.BlockSpec((BLK, H), lambda i: (i, 0)),
            pl.BlockSpec((1, 1, BLK), lambda i: (i, 0, 0)),
            pl.BlockSpec((H, 16), lambda i: (0, 0)),
            pl.BlockSpec((1, 16), lambda i: (0, 0)),
        ],
        out_specs=pl.BlockSpec((G, 16), lambda i: (0, 0)),
        out_shape=jax.ShapeDtypeStruct((G, 16), jnp.float32),
        scratch_shapes=[
            pltpu.VMEM((G, H), jnp.float32),
            pltpu.VMEM((G, H), jnp.float32),
        ],
    )(s, deg2, rp, batch3, wd, bd)


def kernel(x, edge_index, batch, W1, V1, b1, W2, V2, b2, W3, V3, b3, Wd, bd):
    f32 = jnp.float32
    row = edge_index[0]
    col = edge_index[1]
    epad = EPAD - row.shape[0]
    row2d = jnp.concatenate(
        [row, jnp.zeros((epad,), jnp.int32)]).reshape(EROWS, SUB)
    col2d = jnp.concatenate(
        [col, jnp.full((epad,), N, jnp.int32)]).reshape(EROWS, SUB)
    tot = EPAD // CHUNK
    rc3 = jnp.concatenate(
        [row2d.reshape(tot, KSUB, SUB), col2d.reshape(tot, KSUB, SUB)],
        axis=1)

    x_pad = jnp.zeros((NP, D_IN), f32).at[:N].set(x)
    batch3 = jnp.full((NP,), G, jnp.int32).at[:N].set(batch).reshape(
        NBLK, 1, BLK)
    ones16 = jnp.ones((SUB, DEGW), f32)
    z16 = jnp.zeros((NP, DEGW), f32)
    z2 = jnp.zeros((NP, H), jnp.bfloat16)
    wd_pad = jnp.zeros((H, 16), f32).at[:, :D_OUT].set(Wd)
    bd_pad = jnp.zeros((1, 16), f32).at[0, :D_OUT].set(bd)

    deg2 = _deg_call(col2d, z16, ones16)[:, :, 0]

    t1, r = _first_call(x_pad, W1, V1, b1.reshape(1, H))
    tp = _scale_call(t1, deg2)
    s = _edge_call(tp, rc3, z2)
    tp, r = _mid_call(s, deg2, r, W2, V2, b2.reshape(1, H))
    s = _edge_call(tp, rc3, z2)
    tp, r = _mid_call(s, deg2, r, W3, V3, b3.reshape(1, H))
    s = _edge_call(tp, rc3, z2)
    out = _final_call(s, deg2, r, batch3, wd_pad, bd_pad)
    return out[:, :D_OUT]


# speedup vs baseline: 1.4188x; 1.4188x over previous
import functools

import jax
import jax.numpy as jnp
from jax import lax
from jax.experimental import pallas as pl
from jax.experimental.pallas import tpu as pltpu
from jax.experimental.pallas import tpu_sc as plsc

N = 50000
H = 32
D_IN = 128
G = 64
D_OUT = 10

BLK = 1024
NBLK = 49
NP = BLK * NBLK
ROWS_PER_TILE_NODES = NP // 16

NC = 2
NS = 16
SUB = 128
KSUB = 4
CHUNK = SUB * KSUB
NCHUNK = 98
EPAD = NC * NS * NCHUNK * CHUNK
EROWS = EPAD // SUB


def _mesh():
    return plsc.VectorSubcoreMesh(core_axis_name="c", subcore_axis_name="s")


_SC_PARAMS = pltpu.CompilerParams(use_tc_tiling_on_sc=False)


DEGW = 16
DEG_RPI = 8
DEG_ITERS = EROWS // (NC * NS) // DEG_RPI


def _deg_body(col2d, z16, ones_h, out, colv, ones_v, sem, acc):
    cid = lax.axis_index("c")
    sid = lax.axis_index("s")
    r0 = sid * ROWS_PER_TILE_NODES
    pltpu.sync_copy(z16.at[pl.ds(r0, ROWS_PER_TILE_NODES)],
                    acc.at[pl.ds(r0, ROWS_PER_TILE_NODES)])
    pltpu.sync_copy(ones_h, ones_v)
    plsc.subcore_barrier()

    base = (cid * NS + sid) * DEG_ITERS * DEG_RPI

    def body(i, carry):
        off = base + i * DEG_RPI
        pltpu.sync_copy(col2d.at[pl.ds(off, DEG_RPI)], colv)
        for j in range(DEG_RPI):
            pltpu.async_copy(ones_v, acc.at[colv.at[j]], sem, add=True)
        for j in range(DEG_RPI):
            pltpu.make_async_copy(ones_v, acc.at[colv.at[j]], sem).wait()
        return carry

    lax.fori_loop(0, DEG_ITERS, body, 0)
    plsc.subcore_barrier()
    pltpu.sync_copy(acc.at[pl.ds(r0, ROWS_PER_TILE_NODES)],
                    out.at[cid, pl.ds(r0, ROWS_PER_TILE_NODES)])


def _deg_call(col2d, z16, ones_h):
    return pl.kernel(
        _deg_body,
        out_type=jax.ShapeDtypeStruct((NC, NP, DEGW), jnp.float32),
        mesh=_mesh(),
        scratch_types=[
            pltpu.VMEM((DEG_RPI, SUB), jnp.int32),
            pltpu.VMEM((SUB, DEGW), jnp.float32),
            pltpu.SemaphoreType.DMA,
            pltpu.VMEM_SHARED((NP, DEGW), jnp.float32),
        ],
        compiler_params=_SC_PARAMS,
    )(col2d, z16, ones_h)


NPAIR = NCHUNK // 2


def _edge_body(tp, rc3, z2, out,
               idxA, idxB, msgA, msgB, semA, semB, semSA, semSB, acc):
    cid = lax.axis_index("c")
    sid = lax.axis_index("s")
    r0 = sid * ROWS_PER_TILE_NODES
    pltpu.sync_copy(z2.at[pl.ds(r0, ROWS_PER_TILE_NODES)],
                    acc.at[pl.ds(r0, ROWS_PER_TILE_NODES)])
    plsc.subcore_barrier()

    base = (cid * NS + sid) * NCHUNK

    def fire_g(idx, msg, sem):
        for j in range(KSUB):
            pltpu.async_copy(tp.at[idx.at[j]], msg.at[j], sem)

    def drain_g(idx, msg, sem):
        for j in range(KSUB):
            pltpu.make_async_copy(tp.at[idx.at[j]], msg.at[j], sem).wait()

    def fire_s(idx, msg, sem):
        for j in range(KSUB):
            pltpu.async_copy(msg.at[j], acc.at[idx.at[KSUB + j]], sem,
                             add=True)

    def drain_s(idx, msg, sem):
        for j in range(KSUB):
            pltpu.make_async_copy(msg.at[j], acc.at[idx.at[KSUB + j]],
                                  sem).wait()

    pltpu.sync_copy(rc3.at[base], idxA)
    fire_g(idxA, msgA, semA)

    def pair(u, carry):
        c1 = base + 2 * u + 1
        pltpu.sync_copy(rc3.at[c1], idxB)
        fire_g(idxB, msgB, semB)
        drain_g(idxA, msgA, semA)
        fire_s(idxA, msgA, semSA)
        drain_s(idxA, msgA, semSA)
        pltpu.sync_copy(rc3.at[c1 + 1], idxA)
        fire_g(idxA, msgA, semA)
        drain_g(idxB, msgB, semB)
        fire_s(idxB, msgB, semSB)
        drain_s(idxB, msgB, semSB)
        return carry

    lax.fori_loop(0, NPAIR - 1, pair, 0)

    c1 = base + NCHUNK - 1
    pltpu.sync_copy(rc3.at[c1], idxB)
    fire_g(idxB, msgB, semB)
    drain_g(idxA, msgA, semA)
    fire_s(idxA, msgA, semSA)
    drain_s(idxA, msgA, semSA)
    drain_g(idxB, msgB, semB)
    fire_s(idxB, msgB, semSB)
    drain_s(idxB, msgB, semSB)

    plsc.subcore_barrier()
    pltpu.sync_copy(acc.at[pl.ds(r0, ROWS_PER_TILE_NODES)],
                    out.at[cid, pl.ds(r0, ROWS_PER_TILE_NODES)])


def _edge_call(tp, rc3, z2):
    return pl.kernel(
        _edge_body,
        out_type=jax.ShapeDtypeStruct((NC, NP, H), jnp.bfloat16),
        mesh=_mesh(),
        scratch_types=[
            pltpu.VMEM((2 * KSUB, SUB), jnp.int32),
            pltpu.VMEM((2 * KSUB, SUB), jnp.int32),
            pltpu.VMEM((KSUB, SUB, H), jnp.bfloat16),
            pltpu.VMEM((KSUB, SUB, H), jnp.bfloat16),
            pltpu.SemaphoreType.DMA,
            pltpu.SemaphoreType.DMA,
            pltpu.SemaphoreType.DMA,
            pltpu.SemaphoreType.DMA,
            pltpu.VMEM_SHARED((NP, H), jnp.bfloat16),
        ],
        compiler_params=_SC_PARAMS,
    )(tp, rc3, z2)


def _dinv(deg_ref):
    deg = deg_ref[0, :] + deg_ref[1, :]
    return jnp.where(deg > 0, lax.rsqrt(deg), 0.0)


def _first_body(x_ref, w_ref, v_ref, b_ref, t_ref, r_ref):
    x = x_ref[...]
    t_ref[...] = jnp.dot(x, w_ref[...], preferred_element_type=jnp.float32)
    r_ref[...] = jnp.dot(x, v_ref[...],
                         preferred_element_type=jnp.float32) + b_ref[...]


def _first_call(x, w, v, b):
    return pl.pallas_call(
        _first_body,
        grid=(NBLK,),
        in_specs=[
            pl.BlockSpec((BLK, D_IN), lambda i: (i, 0)),
            pl.BlockSpec((D_IN, H), lambda i: (0, 0)),
            pl.BlockSpec((D_IN, H), lambda i: (0, 0)),
            pl.BlockSpec((1, H), lambda i: (0, 0)),
        ],
        out_specs=[
            pl.BlockSpec((BLK, H), lambda i: (i, 0)),
            pl.BlockSpec((BLK, H), lambda i: (i, 0)),
        ],
        out_shape=[
            jax.ShapeDtypeStruct((NP, H), jnp.float32),
            jax.ShapeDtypeStruct((NP, H), jnp.float32),
        ],
    )(x, w, v, b)


def _scale_body(t_ref, deg_ref, tp_ref):
    dinv = _dinv(deg_ref)
    tp_ref[...] = (t_ref[...] * dinv[:, None]).astype(jnp.bfloat16)


def _scale_call(t, deg2):
    return pl.pallas_call(
        _scale_body,
        grid=(NBLK,),
        in_specs=[
            pl.BlockSpec((BLK, H), lambda i: (i, 0)),
            pl.BlockSpec((NC, BLK), lambda i: (0, i)),
        ],
        out_specs=pl.BlockSpec((BLK, H), lambda i: (i, 0)),
        out_shape=jax.ShapeDtypeStruct((NP, H), jnp.bfloat16),
    )(t, deg2)


def _mid_body(s_ref, deg_ref, rp_ref, w_ref, v_ref, b_ref, tp_ref, r_ref):
    dinv = _dinv(deg_ref)
    s = s_ref[0].astype(jnp.float32) + s_ref[1].astype(jnp.float32)
    h = jnp.maximum(dinv[:, None] * s + rp_ref[...], 0.0)
    t = jnp.dot(h, w_ref[...], preferred_element_type=jnp.float32)
    tp_ref[...] = (t * dinv[:, None]).astype(jnp.bfloat16)
    r_ref[...] = jnp.dot(h, v_ref[...],
                         preferred_element_type=jnp.float32) + b_ref[...]


def _mid_call(s, deg2, rp, w, v, b):
    return pl.pallas_call(
        _mid_body,
        grid=(NBLK,),
        in_specs=[
            pl.BlockSpec((NC, BLK, H), lambda i: (0, i, 0)),
            pl.BlockSpec((NC, BLK), lambda i: (0, i)),
            pl.BlockSpec((BLK, H), lambda i: (i, 0)),
            pl.BlockSpec((H, H), lambda i: (0, 0)),
            pl.BlockSpec((H, H), lambda i: (0, 0)),
            pl.BlockSpec((1, H), lambda i: (0, 0)),
        ],
        out_specs=[
            pl.BlockSpec((BLK, H), lambda i: (i, 0)),
            pl.BlockSpec((BLK, H), lambda i: (i, 0)),
        ],
        out_shape=[
            jax.ShapeDtypeStruct((NP, H), jnp.bfloat16),
            jax.ShapeDtypeStruct((NP, H), jnp.float32),
        ],
    )(s, deg2, rp, w, v, b)


def _final_body(s_ref, deg_ref, rp_ref, batch_ref, wd_ref, bd_ref, out_ref,
                sums, cnt):
    i = pl.program_id(0)

    @pl.when(i == 0)
    def _():
        sums[...] = jnp.zeros_like(sums)
        cnt[...] = jnp.zeros_like(cnt)

    dinv = _dinv(deg_ref)
    s = s_ref[0].astype(jnp.float32) + s_ref[1].astype(jnp.float32)
    h = jnp.maximum(dinv[:, None] * s + rp_ref[...], 0.0)
    b = batch_ref[0, 0, :]
    onehot = (b[:, None] == lax.broadcasted_iota(jnp.int32, (BLK, G), 1)
              ).astype(jnp.float32)
    sums[...] += lax.dot_general(onehot, h, (((0,), (0,)), ((), ())),
                                 preferred_element_type=jnp.float32)
    csum = jnp.sum(onehot, axis=0)
    cnt[...] += jnp.broadcast_to(csum[:, None], cnt.shape)

    @pl.when(i == NBLK - 1)
    def _():
        pooled = sums[...] / jnp.maximum(cnt[...], 1.0)
        logits = jnp.dot(pooled, wd_ref[...],
                         preferred_element_type=jnp.float32) + bd_ref[...]
        out_ref[...] = jax.nn.sigmoid(logits)


def _final_call(s, deg2, rp, batch3, wd, bd):
    return pl.pallas_call(
        _final_body,
        grid=(NBLK,),
        in_specs=[
            pl.BlockSpec((NC, BLK, H), lambda i: (0, i, 0)),
            pl.BlockSpec((NC, BLK), lambda i: (0, i)),
            pl.BlockSpec((BLK, H), lambda i: (i, 0)),
            pl.BlockSpec((1, 1, BLK), lambda i: (i, 0, 0)),
            pl.BlockSpec((H, 16), lambda i: (0, 0)),
            pl.BlockSpec((1, 16), lambda i: (0, 0)),
        ],
        out_specs=pl.BlockSpec((G, 16), lambda i: (0, 0)),
        out_shape=jax.ShapeDtypeStruct((G, 16), jnp.float32),
        scratch_shapes=[
            pltpu.VMEM((G, H), jnp.float32),
            pltpu.VMEM((G, H), jnp.float32),
        ],
    )(s, deg2, rp, batch3, wd, bd)


def kernel(x, edge_index, batch, W1, V1, b1, W2, V2, b2, W3, V3, b3, Wd, bd):
    f32 = jnp.float32
    row = edge_index[0]
    col = edge_index[1]
    epad = EPAD - row.shape[0]
    row2d = jnp.concatenate(
        [row, jnp.zeros((epad,), jnp.int32)]).reshape(EROWS, SUB)
    col2d = jnp.concatenate(
        [col, jnp.full((epad,), N, jnp.int32)]).reshape(EROWS, SUB)
    tot = EPAD // CHUNK
    rc3 = jnp.concatenate(
        [row2d.reshape(tot, KSUB, SUB), col2d.reshape(tot, KSUB, SUB)],
        axis=1)

    x_pad = jnp.zeros((NP, D_IN), f32).at[:N].set(x)
    batch3 = jnp.full((NP,), G, jnp.int32).at[:N].set(batch).reshape(
        NBLK, 1, BLK)
    ones16 = jnp.ones((SUB, DEGW), f32)
    z16 = jnp.zeros((NP, DEGW), f32)
    z2 = jnp.zeros((NP, H), jnp.bfloat16)
    wd_pad = jnp.zeros((H, 16), f32).at[:, :D_OUT].set(Wd)
    bd_pad = jnp.zeros((1, 16), f32).at[0, :D_OUT].set(bd)

    deg2 = _deg_call(col2d, z16, ones16)[:, :, 0]

    t1, r = _first_call(x_pad, W1, V1, b1.reshape(1, H))
    tp = _scale_call(t1, deg2)
    s = _edge_call(tp, rc3, z2)
    tp, r = _mid_call(s, deg2, r, W2, V2, b2.reshape(1, H))
    s = _edge_call(tp, rc3, z2)
    tp, r = _mid_call(s, deg2, r, W3, V3, b3.reshape(1, H))
    s = _edge_call(tp, rc3, z2)
    out = _final_call(s, deg2, r, batch3, wd_pad, bd_pad)
    return out[:, :D_OUT]
